# Initial kernel scaffold; baseline (speedup 1.0000x reference)
import jax
import jax.numpy as jnp
from jax.experimental import pallas as pl


def kernel(support_features, query_features, support_labels, query_labels):
    raise NotImplementedError("write your pallas kernel here")


# speedup vs baseline: 4.6030x; 4.6030x over previous
import functools

import jax
import jax.numpy as jnp
from jax.experimental import pallas as pl
from jax.experimental.pallas import tpu as pltpu

Q = 2048
S = 4096
D = 512
BQ = 256
BS = 512
NI = Q // BQ
NJ = S // BS


def _matcher_kernel(q_ref, s_ref, scls_ref, qcls_ref, out_ref,
                    rmax_ref, rcls_ref):
    i = pl.program_id(0)
    j = pl.program_id(1)

    q = q_ref[...]
    s = s_ref[...]

    sim = jax.lax.dot_general(
        q, s, (((1,), (1,)), ((), ())),
        preferred_element_type=jnp.float32)

    bmax = jnp.max(sim, axis=1, keepdims=True)
    lane = jax.lax.broadcasted_iota(jnp.int32, (BQ, BS), 1)
    bidx = jnp.min(jnp.where(sim == bmax, lane, BS),
                   axis=1, keepdims=True)
    cls_row = scls_ref[0]
    bcls = jnp.sum(jnp.where(lane == bidx, cls_row, 0),
                   axis=1, keepdims=True)

    @pl.when(j == 0)
    def _():
        rmax_ref[...] = bmax
        rcls_ref[...] = bcls

    @pl.when(j > 0)
    def _():
        take = bmax > rmax_ref[...]
        rmax_ref[...] = jnp.where(take, bmax, rmax_ref[...])
        rcls_ref[...] = jnp.where(take, bcls, rcls_ref[...])

    @pl.when(j == NJ - 1)
    def _():
        qcls = qcls_ref[0]
        cnt = jnp.sum((rcls_ref[...] == qcls).astype(jnp.float32))
        prev = jnp.where(i == 0, 0.0, out_ref[0, 0])
        tot = prev + cnt
        out_ref[0, 0] = jnp.where(i == NI - 1, tot / Q, tot)


@functools.partial(jax.jit, static_argnames=())
def kernel(support_features, query_features, support_labels, query_labels):
    qn = query_features / jnp.clip(
        jnp.linalg.norm(query_features, axis=1, keepdims=True), 1e-8)
    sn = support_features / jnp.clip(
        jnp.linalg.norm(support_features, axis=1, keepdims=True), 1e-8)

    scls = support_labels[:, 0].astype(jnp.int32).reshape(NJ, 1, BS)
    qcls = query_labels[:, 0].astype(jnp.int32).reshape(NI, BQ, 1)

    out = pl.pallas_call(
        _matcher_kernel,
        grid=(NI, NJ),
        in_specs=[
            pl.BlockSpec((BQ, D), lambda i, j: (i, 0)),
            pl.BlockSpec((BS, D), lambda i, j: (j, 0)),
            pl.BlockSpec((1, 1, BS), lambda i, j: (j, 0, 0)),
            pl.BlockSpec((1, BQ, 1), lambda i, j: (i, 0, 0)),
        ],
        out_specs=pl.BlockSpec((1, 1), lambda i, j: (0, 0),
                               memory_space=pltpu.SMEM),
        out_shape=jax.ShapeDtypeStruct((1, 1), jnp.float32),
        scratch_shapes=[
            pltpu.VMEM((BQ, 1), jnp.float32),
            pltpu.VMEM((BQ, 1), jnp.int32),
        ],
        compiler_params=pltpu.CompilerParams(
            dimension_semantics=("arbitrary", "arbitrary"),
        ),
    )(qn, sn, scls, qcls)
    return out[0, 0]


# speedup vs baseline: 10.2431x; 2.2253x over previous
import jax
import jax.numpy as jnp
from jax.experimental import pallas as pl
from jax.experimental.pallas import tpu as pltpu

Q = 2048
S = 4096
D = 512
BQ = 256
NI = Q // BQ


def _matcher_kernel(q_ref, s_ref, scode_ref, qcls_ref, out_ref):
    i = pl.program_id(0)

    q = q_ref[...]
    s = s_ref[...]

    sim = jax.lax.dot_general(
        q, s, (((1,), (1,)), ((), ())),
        preferred_element_type=jnp.float32)

    bmax = jnp.max(sim, axis=1, keepdims=True)
    code = jnp.where(sim == bmax, scode_ref[0], jnp.int32(2 ** 30))
    bcode = jnp.min(code, axis=1, keepdims=True)
    bcls = jax.lax.rem(bcode, jnp.int32(64))

    cnt = jnp.sum((bcls == qcls_ref[0]).astype(jnp.float32))
    prev = jnp.where(i == 0, 0.0, out_ref[0, 0])
    tot = prev + cnt
    out_ref[0, 0] = jnp.where(i == NI - 1, tot / Q, tot)


def kernel(support_features, query_features, support_labels, query_labels):
    qn = query_features / jnp.clip(
        jnp.linalg.norm(query_features, axis=1, keepdims=True), 1e-8)
    sn = support_features / jnp.clip(
        jnp.linalg.norm(support_features, axis=1, keepdims=True), 1e-8)

    scls = support_labels[:, 0].astype(jnp.int32)
    scode = (jnp.arange(S, dtype=jnp.int32) * 64 + scls).reshape(1, 1, S)
    qcls = query_labels[:, 0].astype(jnp.int32).reshape(NI, BQ, 1)

    out = pl.pallas_call(
        _matcher_kernel,
        grid=(NI,),
        in_specs=[
            pl.BlockSpec((BQ, D), lambda i: (i, 0)),
            pl.BlockSpec((S, D), lambda i: (0, 0)),
            pl.BlockSpec((1, 1, S), lambda i: (0, 0, 0)),
            pl.BlockSpec((1, BQ, 1), lambda i: (i, 0, 0)),
        ],
        out_specs=pl.BlockSpec((1, 1), lambda i: (0, 0),
                               memory_space=pltpu.SMEM),
        out_shape=jax.ShapeDtypeStruct((1, 1), jnp.float32),
        compiler_params=pltpu.CompilerParams(
            dimension_semantics=("arbitrary",),
        ),
    )(qn, sn, scode, qcls)
    return out[0, 0]


# speedup vs baseline: 10.8711x; 1.0613x over previous
import jax
import jax.numpy as jnp
from jax.experimental import pallas as pl
from jax.experimental.pallas import tpu as pltpu

Q = 2048
S = 4096
D = 512
BQ = 512
NI = Q // BQ


def _matcher_kernel(q_ref, s_ref, scode_ref, qcls_ref, out_ref):
    i = pl.program_id(0)

    q = q_ref[...]
    s = s_ref[...]

    sim = jax.lax.dot_general(
        q, s, (((1,), (1,)), ((), ())),
        preferred_element_type=jnp.float32)

    bmax = jnp.max(sim, axis=1, keepdims=True)
    code = jnp.where(sim == bmax, scode_ref[0], jnp.int32(2 ** 30))
    bcode = jnp.min(code, axis=1, keepdims=True)
    bcls = jax.lax.rem(bcode, jnp.int32(64))

    cnt = jnp.sum((bcls == qcls_ref[0]).astype(jnp.float32))
    prev = jnp.where(i == 0, 0.0, out_ref[0, 0])
    tot = prev + cnt
    out_ref[0, 0] = jnp.where(i == NI - 1, tot / Q, tot)


def kernel(support_features, query_features, support_labels, query_labels):
    qn = query_features / jnp.clip(
        jnp.linalg.norm(query_features, axis=1, keepdims=True), 1e-8)
    sn = support_features / jnp.clip(
        jnp.linalg.norm(support_features, axis=1, keepdims=True), 1e-8)

    scls = support_labels[:, 0].astype(jnp.int32)
    scode = (jnp.arange(S, dtype=jnp.int32) * 64 + scls).reshape(1, 1, S)
    qcls = query_labels[:, 0].astype(jnp.int32).reshape(NI, BQ, 1)

    out = pl.pallas_call(
        _matcher_kernel,
        grid=(NI,),
        in_specs=[
            pl.BlockSpec((BQ, D), lambda i: (i, 0)),
            pl.BlockSpec((S, D), lambda i: (0, 0)),
            pl.BlockSpec((1, 1, S), lambda i: (0, 0, 0)),
            pl.BlockSpec((1, BQ, 1), lambda i: (i, 0, 0)),
        ],
        out_specs=pl.BlockSpec((1, 1), lambda i: (0, 0),
                               memory_space=pltpu.SMEM),
        out_shape=jax.ShapeDtypeStruct((1, 1), jnp.float32),
        compiler_params=pltpu.CompilerParams(
            dimension_semantics=("arbitrary",),
        ),
    )(qn, sn, scode, qcls)
    return out[0, 0]


# speedup vs baseline: 11.9616x; 1.1003x over previous
import jax
import jax.numpy as jnp
from jax.experimental import pallas as pl
from jax.experimental.pallas import tpu as pltpu

Q = 2048
S = 4096
D = 512
BQ = 512
NI = Q // BQ


def _matcher_kernel(q_ref, s_ref, qnorm_ref, snorm_ref, scode_ref, qcls_ref,
                    out_ref, sn_ref):
    i = pl.program_id(0)

    @pl.when(i == 0)
    def _():
        sn_ref[...] = s_ref[...] / snorm_ref[...]

    q = q_ref[...] / qnorm_ref[...]

    sim = jax.lax.dot_general(
        q, sn_ref[...], (((1,), (1,)), ((), ())),
        preferred_element_type=jnp.float32)

    bmax = jnp.max(sim, axis=1, keepdims=True)
    code = jnp.where(sim == bmax, scode_ref[0], jnp.int32(2 ** 30))
    bcode = jnp.min(code, axis=1, keepdims=True)
    bcls = jax.lax.rem(bcode, jnp.int32(64))

    cnt = jnp.sum((bcls == qcls_ref[0]).astype(jnp.float32))
    prev = jnp.where(i == 0, 0.0, out_ref[0, 0])
    tot = prev + cnt
    out_ref[0, 0] = jnp.where(i == NI - 1, tot / Q, tot)


def kernel(support_features, query_features, support_labels, query_labels):
    qnorm = jnp.clip(
        jnp.linalg.norm(query_features, axis=1, keepdims=True), 1e-8)
    snorm = jnp.clip(
        jnp.linalg.norm(support_features, axis=1, keepdims=True), 1e-8)

    scls = support_labels[:, 0].astype(jnp.int32)
    scode = (jnp.arange(S, dtype=jnp.int32) * 64 + scls).reshape(1, 1, S)
    qcls = query_labels[:, 0].astype(jnp.int32).reshape(NI, BQ, 1)

    out = pl.pallas_call(
        _matcher_kernel,
        grid=(NI,),
        in_specs=[
            pl.BlockSpec((BQ, D), lambda i: (i, 0)),
            pl.BlockSpec((S, D), lambda i: (0, 0)),
            pl.BlockSpec((BQ, 1), lambda i: (i, 0)),
            pl.BlockSpec((S, 1), lambda i: (0, 0)),
            pl.BlockSpec((1, 1, S), lambda i: (0, 0, 0)),
            pl.BlockSpec((1, BQ, 1), lambda i: (i, 0, 0)),
        ],
        out_specs=pl.BlockSpec((1, 1), lambda i: (0, 0),
                               memory_space=pltpu.SMEM),
        out_shape=jax.ShapeDtypeStruct((1, 1), jnp.float32),
        scratch_shapes=[pltpu.VMEM((S, D), jnp.float32)],
        compiler_params=pltpu.CompilerParams(
            dimension_semantics=("arbitrary",),
        ),
    )(query_features, support_features, qnorm, snorm, scode, qcls)
    return out[0, 0]


# speedup vs baseline: 12.3008x; 1.0284x over previous
import jax
import jax.numpy as jnp
from jax.experimental import pallas as pl
from jax.experimental.pallas import tpu as pltpu

Q = 2048
S = 4096
D = 512
BQ = 1024
NI = Q // BQ


def _matcher_kernel(q_ref, s_ref, qnorm_ref, snorm_ref, scode_ref, qlab_ref,
                    out_ref, sn_ref):
    i = pl.program_id(0)

    @pl.when(i == 0)
    def _():
        sn_ref[...] = s_ref[...] / snorm_ref[...]

    q = q_ref[...] / qnorm_ref[...]

    sim = jax.lax.dot_general(
        q, sn_ref[...], (((1,), (1,)), ((), ())),
        preferred_element_type=jnp.float32)

    bmax = jnp.max(sim, axis=1, keepdims=True)
    code = jnp.where(sim == bmax, scode_ref[0], jnp.int32(2 ** 30))
    bcode = jnp.min(code, axis=1, keepdims=True)
    bcls = jax.lax.rem(bcode, jnp.int32(64))

    qcls = qlab_ref[0][:, 0:1]
    cnt = jnp.sum((bcls == qcls).astype(jnp.float32))
    prev = jnp.where(i == 0, 0.0, out_ref[0, 0])
    tot = prev + cnt
    out_ref[0, 0] = jnp.where(i == NI - 1, tot / Q, tot)


def kernel(support_features, query_features, support_labels, query_labels):
    qnorm = jnp.clip(
        jnp.linalg.norm(query_features, axis=1, keepdims=True), 1e-8)
    snorm = jnp.clip(
        jnp.linalg.norm(support_features, axis=1, keepdims=True), 1e-8)

    scls = support_labels[:, 0].astype(jnp.int32)
    scode = (jnp.arange(S, dtype=jnp.int32) * 64 + scls).reshape(1, 1, S)
    qlab = query_labels.astype(jnp.int32).reshape(NI, BQ, 2)

    out = pl.pallas_call(
        _matcher_kernel,
        grid=(NI,),
        in_specs=[
            pl.BlockSpec((BQ, D), lambda i: (i, 0)),
            pl.BlockSpec((S, D), lambda i: (0, 0)),
            pl.BlockSpec((BQ, 1), lambda i: (i, 0)),
            pl.BlockSpec((S, 1), lambda i: (0, 0)),
            pl.BlockSpec((1, 1, S), lambda i: (0, 0, 0)),
            pl.BlockSpec((1, BQ, 2), lambda i: (i, 0, 0)),
        ],
        out_specs=pl.BlockSpec((1, 1), lambda i: (0, 0),
                               memory_space=pltpu.SMEM),
        out_shape=jax.ShapeDtypeStruct((1, 1), jnp.float32),
        scratch_shapes=[pltpu.VMEM((S, D), jnp.float32)],
        compiler_params=pltpu.CompilerParams(
            dimension_semantics=("arbitrary",),
        ),
    )(query_features, support_features, qnorm, snorm, scode, qlab)
    return out[0, 0]


# speedup vs baseline: 12.3113x; 1.0008x over previous
import jax
import jax.numpy as jnp
from jax.experimental import pallas as pl
from jax.experimental.pallas import tpu as pltpu

Q = 2048
S = 4096
D = 512
BQ = 1024
NI = Q // BQ
NC = 4
CS = S // NC


def _matcher_kernel(q_ref, s_ref, qnorm_ref, snorm_ref, scode_ref, qlab_ref,
                    out_ref, sn_ref):
    i = pl.program_id(0)

    @pl.when(i == 0)
    def _():
        sn_ref[...] = s_ref[...] / snorm_ref[...]

    q = q_ref[...] / qnorm_ref[...]

    sims, maxs, codes = [], [], []
    for c in range(NC):
        sc = sn_ref[pl.ds(c * CS, CS), :]
        sim_c = jax.lax.dot_general(
            q, sc, (((1,), (1,)), ((), ())),
            preferred_element_type=jnp.float32)
        code_c = scode_ref[0][:, c * CS:(c + 1) * CS]
        bmax_c = jnp.max(sim_c, axis=1, keepdims=True)
        bcode_c = jnp.min(
            jnp.where(sim_c == bmax_c, code_c, jnp.int32(2 ** 30)),
            axis=1, keepdims=True)
        maxs.append(bmax_c)
        codes.append(bcode_c)

    bmax = maxs[0]
    for c in range(1, NC):
        bmax = jnp.maximum(bmax, maxs[c])
    bcode = jnp.int32(2 ** 30)
    for c in range(NC):
        bcode = jnp.minimum(
            bcode, jnp.where(maxs[c] == bmax, codes[c], jnp.int32(2 ** 30)))
    bcls = jax.lax.rem(bcode, jnp.int32(64))

    qcls = qlab_ref[0][:, 0:1]
    cnt = jnp.sum((bcls == qcls).astype(jnp.float32))
    prev = jnp.where(i == 0, 0.0, out_ref[0, 0])
    tot = prev + cnt
    out_ref[0, 0] = jnp.where(i == NI - 1, tot / Q, tot)


def kernel(support_features, query_features, support_labels, query_labels):
    qnorm = jnp.clip(
        jnp.linalg.norm(query_features, axis=1, keepdims=True), 1e-8)
    snorm = jnp.clip(
        jnp.linalg.norm(support_features, axis=1, keepdims=True), 1e-8)

    scls = support_labels[:, 0].astype(jnp.int32)
    scode = (jnp.arange(S, dtype=jnp.int32) * 64 + scls).reshape(1, 1, S)
    qlab = query_labels.astype(jnp.int32).reshape(NI, BQ, 2)

    out = pl.pallas_call(
        _matcher_kernel,
        grid=(NI,),
        in_specs=[
            pl.BlockSpec((BQ, D), lambda i: (i, 0)),
            pl.BlockSpec((S, D), lambda i: (0, 0)),
            pl.BlockSpec((BQ, 1), lambda i: (i, 0)),
            pl.BlockSpec((S, 1), lambda i: (0, 0)),
            pl.BlockSpec((1, 1, S), lambda i: (0, 0, 0)),
            pl.BlockSpec((1, BQ, 2), lambda i: (i, 0, 0)),
        ],
        out_specs=pl.BlockSpec((1, 1), lambda i: (0, 0),
                               memory_space=pltpu.SMEM),
        out_shape=jax.ShapeDtypeStruct((1, 1), jnp.float32),
        scratch_shapes=[pltpu.VMEM((S, D), jnp.float32)],
        compiler_params=pltpu.CompilerParams(
            dimension_semantics=("arbitrary",),
        ),
    )(query_features, support_features, qnorm, snorm, scode, qlab)
    return out[0, 0]
